# baseline (speedup 1.0000x reference)
import functools

import jax
import jax.numpy as jnp
from jax import lax
from jax.experimental import pallas as pl
from jax.experimental.pallas import tpu as pltpu
from jax.experimental.pallas import tpu_sc as plsc

N = 10000
E = 320000
IN_C = 128
OUT_C = 16

NC = 2
NS = 16
NW = NC * NS
CH = 128
EPT = 10240
EP = NW * EPT
NJ = EPT // CH
NP = 10112
RPT = NP // NS

_EPS = jax.random.normal(jax.random.key(42), (N, OUT_C), dtype=jnp.float32)


DW = 8


def _deg_body(dst_hbm, ones_hbm, z8_hbm, out_hbm, idx_v, ones_v, acc, sem):
    c = lax.axis_index("c")
    s = lax.axis_index("s")
    wid = s * NC + c
    pltpu.sync_copy(ones_hbm, ones_v)
    pltpu.sync_copy(z8_hbm.at[pl.ds(s * RPT, RPT)], acc.at[pl.ds(s * RPT, RPT)])
    pltpu.sync_copy(dst_hbm.at[pl.ds(wid * NJ, NJ)], idx_v)
    plsc.subcore_barrier()

    def body(j, carry):
        pltpu.sync_copy(ones_v, acc.at[idx_v.at[j]], add=True)
        return carry

    lax.fori_loop(0, NJ, body, 0)
    plsc.subcore_barrier()
    pltpu.sync_copy(acc.at[pl.ds(s * RPT, RPT)],
                    out_hbm.at[c, pl.ds(s * RPT, RPT)])


@functools.cache
def _deg_call():
    mesh = plsc.VectorSubcoreMesh(core_axis_name="c", subcore_axis_name="s",
                                  num_cores=NC, num_subcores=NS)
    return pl.kernel(
        _deg_body,
        out_type=jax.ShapeDtypeStruct((NC, NP, DW), jnp.float32),
        mesh=mesh,
        compiler_params=pltpu.CompilerParams(use_tc_tiling_on_sc=False),
        scratch_types=[
            pltpu.VMEM((NJ, CH), jnp.int32),
            pltpu.VMEM((CH, DW), jnp.float32),
            pltpu.VMEM_SHARED((NP, DW), jnp.float32),
            pltpu.SemaphoreType.DMA,
        ],
    )


NSLOT = 4


def _prop_body(rows_hbm, six_hbm, dix_hbm, z32_hbm, out_hbm,
               sidx, didx, rbs, acc, tbl, gsems, ssems):
    c = lax.axis_index("c")
    s = lax.axis_index("s")
    wid = s * NC + c
    pltpu.sync_copy(six_hbm.at[pl.ds(wid * NJ, NJ)], sidx)
    pltpu.sync_copy(dix_hbm.at[pl.ds(wid * NJ, NJ)], didx)
    rows_slc = pl.ds(s * RPT, RPT)
    pltpu.sync_copy(rows_hbm.at[rows_slc], tbl.at[rows_slc])
    pltpu.sync_copy(z32_hbm.at[rows_slc], acc.at[rows_slc])
    plsc.subcore_barrier()
    for b in range(NSLOT):
        pltpu.async_copy(tbl.at[sidx.at[b]], rbs[b], gsems[b])

    def body(i, carry):
        j0 = NSLOT * i
        for b in range(NSLOT):
            pltpu.make_async_copy(tbl.at[sidx.at[j0 + b]], rbs[b],
                                  gsems[b]).wait()
            pltpu.async_copy(rbs[b], acc.at[didx.at[j0 + b]], ssems[b],
                             add=True)

        @pl.when(i + 1 < NJ // NSLOT)
        def _refill():
            for b in range(NSLOT):
                pltpu.make_async_copy(rbs[b], acc.at[didx.at[j0 + b]],
                                      ssems[b]).wait()
                pltpu.async_copy(tbl.at[sidx.at[j0 + NSLOT + b]], rbs[b],
                                 gsems[b])

        return carry

    lax.fori_loop(0, NJ // NSLOT, body, 0)
    for b in range(NSLOT):
        pltpu.make_async_copy(rbs[b], acc.at[didx.at[NJ - NSLOT + b]],
                              ssems[b]).wait()
    plsc.subcore_barrier()
    pltpu.sync_copy(acc.at[rows_slc], out_hbm.at[c, rows_slc])


@functools.cache
def _prop_call():
    mesh = plsc.VectorSubcoreMesh(core_axis_name="c", subcore_axis_name="s",
                                  num_cores=NC, num_subcores=NS)
    return pl.kernel(
        _prop_body,
        out_type=jax.ShapeDtypeStruct((NC, NP, 32), jnp.float32),
        mesh=mesh,
        compiler_params=pltpu.CompilerParams(use_tc_tiling_on_sc=False),
        scratch_types=[
            pltpu.VMEM((NJ, CH), jnp.int32),
            pltpu.VMEM((NJ, CH), jnp.int32),
            [pltpu.VMEM((CH, 32), jnp.float32) for _ in range(NSLOT)],
            pltpu.VMEM_SHARED((NP, 32), jnp.float32),
            pltpu.VMEM_SHARED((NP, 32), jnp.float32),
            [pltpu.SemaphoreType.DMA for _ in range(NSLOT)],
            [pltpu.SemaphoreType.DMA for _ in range(NSLOT)],
        ],
    )


def _mm1_body(x_ref, w_ref, o_ref):
    o_ref[0:N, :] = jnp.dot(x_ref[...], w_ref[...],
                            preferred_element_type=jnp.float32)
    o_ref[N:NP, :] = jnp.zeros((NP - N, 2 * OUT_C), jnp.float32)


def _scale_body(cnt_ref, h0_ref, hs_ref, dinv_ref, idg_ref):
    deg = cnt_ref[0, :, 0:1] + cnt_ref[1, :, 0:1] + 1.0
    dinv = lax.rsqrt(deg)
    dinv_ref[...] = dinv
    idg_ref[...] = 1.0 / deg
    hs_ref[...] = h0_ref[...] * dinv


def _layer1_body(acc_ref, dinv_ref, idg_ref, h0_ref, b1_ref, w23_ref,
                 g_ref, gs_ref):
    ph = dinv_ref[...] * (acc_ref[0] + acc_ref[1]) \
        + h0_ref[...] * idg_ref[...] + b1_ref[...]
    h = jnp.maximum(ph, 0.0)
    g = jnp.dot(h, w23_ref[...], preferred_element_type=jnp.float32)
    g_ref[...] = g
    gs_ref[...] = g * dinv_ref[...]


def _final_body(acc_ref, dinv_ref, idg_ref, g_ref, b23_ref, eps_ref, z_ref):
    pg = dinv_ref[...] * (acc_ref[0] + acc_ref[1]) \
        + g_ref[...] * idg_ref[...] + b23_ref[...]
    mu = pg[:N, :OUT_C]
    ls = jnp.minimum(pg[:N, OUT_C:], 10.0)
    z_ref[...] = mu + eps_ref[...] * jnp.exp(ls)


def kernel(x, edge_index, W1, b1, W2, b2, W3, b3):
    f32 = jnp.float32
    src = edge_index[0]
    dst = edge_index[1]
    pad = EP - E
    src_p = jnp.concatenate([src, jnp.zeros((pad,), jnp.int32)])
    dst_p = jnp.concatenate([dst, jnp.full((pad,), N, jnp.int32)])
    six = src_p.reshape(NW * NJ, CH)
    dix = dst_p.reshape(NW * NJ, CH)
    z8 = jnp.zeros((NP, DW), f32)
    ones8 = jnp.concatenate(
        [jnp.ones((CH, 1), f32), jnp.zeros((CH, DW - 1), f32)], axis=1)
    z32 = jnp.zeros((NP, 32), f32)
    w23 = jnp.concatenate([W2, W3], axis=1)
    b23 = jnp.concatenate([b2, b3]).reshape(1, 32)
    b1r = b1.reshape(1, 32)

    counts = _deg_call()(dix, ones8, z8)

    h0 = pl.pallas_call(
        _mm1_body,
        out_shape=jax.ShapeDtypeStruct((NP, 32), f32),
    )(x, W1)

    hs, dinv, idg = pl.pallas_call(
        _scale_body,
        out_shape=[
            jax.ShapeDtypeStruct((NP, 32), f32),
            jax.ShapeDtypeStruct((NP, 1), f32),
            jax.ShapeDtypeStruct((NP, 1), f32),
        ],
    )(counts, h0)

    acc1 = _prop_call()(hs, six, dix, z32)

    g, gs = pl.pallas_call(
        _layer1_body,
        out_shape=[
            jax.ShapeDtypeStruct((NP, 32), f32),
            jax.ShapeDtypeStruct((NP, 32), f32),
        ],
    )(acc1, dinv, idg, h0, b1r, w23)

    acc2 = _prop_call()(gs, six, dix, z32)

    z = pl.pallas_call(
        _final_body,
        out_shape=jax.ShapeDtypeStruct((N, OUT_C), f32),
    )(acc2, dinv, idg, g, b23, _EPS)
    return z


# speedup vs baseline: 1.0477x; 1.0477x over previous
import functools

import jax
import jax.numpy as jnp
from jax import lax
from jax.experimental import pallas as pl
from jax.experimental.pallas import tpu as pltpu
from jax.experimental.pallas import tpu_sc as plsc

N = 10000
E = 320000
IN_C = 128
OUT_C = 16

NC = 2
NS = 16
NW = NC * NS
CH = 128
EPT = 10240
EP = NW * EPT
NJ = EPT // CH
NP = 10112
RPT = NP // NS

_EPS = jax.random.normal(jax.random.key(42), (N, OUT_C), dtype=jnp.float32)


DW = 8


def _deg_body(dst_hbm, ones_hbm, z8_hbm, out_hbm, idx_v, ones_v, acc, sem):
    c = lax.axis_index("c")
    s = lax.axis_index("s")
    wid = s * NC + c
    pltpu.sync_copy(ones_hbm, ones_v)
    pltpu.sync_copy(z8_hbm.at[pl.ds(s * RPT, RPT)], acc.at[pl.ds(s * RPT, RPT)])
    pltpu.sync_copy(dst_hbm.at[pl.ds(wid * NJ, NJ)], idx_v)
    plsc.subcore_barrier()

    def body(j, carry):
        pltpu.sync_copy(ones_v, acc.at[idx_v.at[j]], add=True)
        return carry

    lax.fori_loop(0, NJ, body, 0)
    plsc.subcore_barrier()
    pltpu.sync_copy(acc.at[pl.ds(s * RPT, RPT)],
                    out_hbm.at[c, pl.ds(s * RPT, RPT)])


@functools.cache
def _deg_call():
    mesh = plsc.VectorSubcoreMesh(core_axis_name="c", subcore_axis_name="s",
                                  num_cores=NC, num_subcores=NS)
    return pl.kernel(
        _deg_body,
        out_type=jax.ShapeDtypeStruct((NC, NP, DW), jnp.float32),
        mesh=mesh,
        compiler_params=pltpu.CompilerParams(use_tc_tiling_on_sc=False),
        scratch_types=[
            pltpu.VMEM((NJ, CH), jnp.int32),
            pltpu.VMEM((CH, DW), jnp.float32),
            pltpu.VMEM_SHARED((NP, DW), jnp.float32),
            pltpu.SemaphoreType.DMA,
        ],
    )


NSLOT = 4


def _prop_body(rows_hbm, six_hbm, dix_hbm, z32_hbm, out_hbm,
               sidx, didx, rbs, acc, tbl, gsems, ssems):
    c = lax.axis_index("c")
    s = lax.axis_index("s")
    wid = s * NC + c
    pltpu.sync_copy(six_hbm.at[pl.ds(wid * NJ, NJ)], sidx)
    pltpu.sync_copy(dix_hbm.at[pl.ds(wid * NJ, NJ)], didx)
    rows_slc = pl.ds(s * RPT, RPT)
    pltpu.sync_copy(rows_hbm.at[rows_slc], tbl.at[rows_slc])
    pltpu.sync_copy(z32_hbm.at[rows_slc], acc.at[rows_slc])
    plsc.subcore_barrier()
    for b in range(NSLOT):
        pltpu.async_copy(tbl.at[sidx.at[b]], rbs[b], gsems[b])

    def body(i, carry):
        j0 = NSLOT * i
        for b in range(NSLOT):
            pltpu.make_async_copy(tbl.at[sidx.at[j0 + b]], rbs[b],
                                  gsems[b]).wait()
            pltpu.async_copy(rbs[b], acc.at[didx.at[j0 + b]], ssems[b],
                             add=True)

        @pl.when(i + 1 < NJ // NSLOT)
        def _refill():
            for b in range(NSLOT):
                pltpu.make_async_copy(rbs[b], acc.at[didx.at[j0 + b]],
                                      ssems[b]).wait()
                pltpu.async_copy(tbl.at[sidx.at[j0 + NSLOT + b]], rbs[b],
                                 gsems[b])

        return carry

    lax.fori_loop(0, NJ // NSLOT, body, 0)
    for b in range(NSLOT):
        pltpu.make_async_copy(rbs[b], acc.at[didx.at[NJ - NSLOT + b]],
                              ssems[b]).wait()
    plsc.subcore_barrier()
    pltpu.sync_copy(acc.at[rows_slc], out_hbm.at[c, rows_slc])


@functools.cache
def _prop_call():
    mesh = plsc.VectorSubcoreMesh(core_axis_name="c", subcore_axis_name="s",
                                  num_cores=NC, num_subcores=NS)
    return pl.kernel(
        _prop_body,
        out_type=jax.ShapeDtypeStruct((NC, NP, 32), jnp.float32),
        mesh=mesh,
        compiler_params=pltpu.CompilerParams(use_tc_tiling_on_sc=False),
        scratch_types=[
            pltpu.VMEM((NJ, CH), jnp.int32),
            pltpu.VMEM((NJ, CH), jnp.int32),
            [pltpu.VMEM((CH, 32), jnp.float32) for _ in range(NSLOT)],
            pltpu.VMEM_SHARED((NP, 32), jnp.float32),
            pltpu.VMEM_SHARED((NP, 32), jnp.float32),
            [pltpu.SemaphoreType.DMA for _ in range(NSLOT)],
            [pltpu.SemaphoreType.DMA for _ in range(NSLOT)],
        ],
    )


def _mm1_body(x_ref, w_ref, o_ref):
    o_ref[0:N, :] = jnp.dot(x_ref[...], w_ref[...],
                            preferred_element_type=jnp.float32)
    o_ref[N:NP, :] = jnp.zeros((NP - N, 2 * OUT_C), jnp.float32)


def _scale_body(cnt_ref, h0_ref, hs_ref, dinv_ref):
    deg = cnt_ref[0, :, 0:1] + cnt_ref[1, :, 0:1] + 1.0
    dinv = lax.rsqrt(deg)
    dinv_ref[...] = dinv * jnp.ones((1, 2 * OUT_C), jnp.float32)
    hs_ref[...] = h0_ref[...] * dinv


def _layer1_body(acc_ref, dinv_ref, hs_ref, b1_ref, w23_ref, gs_ref):
    ph = dinv_ref[...] * (acc_ref[0] + acc_ref[1] + hs_ref[...]) \
        + b1_ref[...]
    h = jnp.maximum(ph, 0.0)
    g = jnp.dot(h, w23_ref[...], preferred_element_type=jnp.float32)
    gs_ref[...] = g * dinv_ref[...]


def _final_body(acc_ref, dinv_ref, gs_ref, b23_ref, eps_ref, z_ref):
    pg = dinv_ref[...] * (acc_ref[0] + acc_ref[1] + gs_ref[...]) \
        + b23_ref[...]
    mu = pg[:N, :OUT_C]
    ls = jnp.minimum(pg[:N, OUT_C:], 10.0)
    z_ref[...] = mu + eps_ref[...] * jnp.exp(ls)


def kernel(x, edge_index, W1, b1, W2, b2, W3, b3):
    f32 = jnp.float32
    src = edge_index[0]
    dst = edge_index[1]
    pad = EP - E
    src_p = jnp.concatenate([src, jnp.zeros((pad,), jnp.int32)])
    dst_p = jnp.concatenate([dst, jnp.full((pad,), N, jnp.int32)])
    six = src_p.reshape(NW * NJ, CH)
    dix = dst_p.reshape(NW * NJ, CH)
    z8 = jnp.zeros((NP, DW), f32)
    ones8 = jnp.concatenate(
        [jnp.ones((CH, 1), f32), jnp.zeros((CH, DW - 1), f32)], axis=1)
    z32 = jnp.zeros((NP, 32), f32)
    w23 = jnp.concatenate([W2, W3], axis=1)
    b23 = jnp.concatenate([b2, b3]).reshape(1, 32)
    b1r = b1.reshape(1, 32)

    counts = _deg_call()(dix, ones8, z8)

    h0 = pl.pallas_call(
        _mm1_body,
        out_shape=jax.ShapeDtypeStruct((NP, 32), f32),
    )(x, W1)

    hs, dinv32 = pl.pallas_call(
        _scale_body,
        out_shape=[
            jax.ShapeDtypeStruct((NP, 32), f32),
            jax.ShapeDtypeStruct((NP, 32), f32),
        ],
    )(counts, h0)

    acc1 = _prop_call()(hs, six, dix, z32)

    gs = pl.pallas_call(
        _layer1_body,
        out_shape=jax.ShapeDtypeStruct((NP, 32), f32),
    )(acc1, dinv32, hs, b1r, w23)

    acc2 = _prop_call()(gs, six, dix, z32)

    z = pl.pallas_call(
        _final_body,
        out_shape=jax.ShapeDtypeStruct((N, OUT_C), f32),
    )(acc2, dinv32, gs, b23, _EPS)
    return z


# speedup vs baseline: 1.1816x; 1.1278x over previous
import functools

import jax
import jax.numpy as jnp
from jax import lax
from jax.experimental import pallas as pl
from jax.experimental.pallas import tpu as pltpu
from jax.experimental.pallas import tpu_sc as plsc

N = 10000
E = 320000
IN_C = 128
OUT_C = 16

NC = 2
NS = 16
NW = NC * NS
CH = 128
EC = E // CH
NJ0 = EC // NW
NXT = EC - NW * NJ0
KT = NJ0 + 1
NP = 10112
RPT = NP // NS

_EPS = jax.random.normal(jax.random.key(42), (N, OUT_C), dtype=jnp.float32)


DW = 8


def _deg_body(edge_hbm, ones_hbm, z8_hbm, out_hbm, idx_v, ones_v, acc, sem):
    c = lax.axis_index("c")
    s = lax.axis_index("s")
    wid = s * NC + c
    nj = NJ0 + jnp.where(wid < NXT, 1, 0)
    base = NJ0 * wid + jnp.minimum(wid, NXT)
    pltpu.sync_copy(ones_hbm, ones_v)
    pltpu.sync_copy(z8_hbm.at[pl.ds(s * RPT, RPT)], acc.at[pl.ds(s * RPT, RPT)])
    pltpu.sync_copy(edge_hbm.at[1, pl.ds(base, NJ0)], idx_v.at[pl.ds(0, NJ0)])

    @pl.when(wid < NXT)
    def _extra():
        pltpu.sync_copy(edge_hbm.at[1, pl.ds(base + NJ0, 1)],
                        idx_v.at[pl.ds(NJ0, 1)])

    plsc.subcore_barrier()

    def body(j, carry):
        pltpu.sync_copy(ones_v, acc.at[idx_v.at[j]], add=True)
        return carry

    lax.fori_loop(0, nj, body, 0)
    plsc.subcore_barrier()
    pltpu.sync_copy(acc.at[pl.ds(s * RPT, RPT)],
                    out_hbm.at[c, pl.ds(s * RPT, RPT)])


@functools.cache
def _deg_call():
    mesh = plsc.VectorSubcoreMesh(core_axis_name="c", subcore_axis_name="s",
                                  num_cores=NC, num_subcores=NS)
    return pl.kernel(
        _deg_body,
        out_type=jax.ShapeDtypeStruct((NC, NP, DW), jnp.float32),
        mesh=mesh,
        compiler_params=pltpu.CompilerParams(use_tc_tiling_on_sc=False),
        scratch_types=[
            pltpu.VMEM((KT, CH), jnp.int32),
            pltpu.VMEM((CH, DW), jnp.float32),
            pltpu.VMEM_SHARED((NP, DW), jnp.float32),
            pltpu.SemaphoreType.DMA,
        ],
    )


NSLOT = 2


def _prop_body(rows_hbm, edge_hbm, z32_hbm, out_hbm,
               sidx, didx, rbs, acc, tbl, gsems, ssems):
    c = lax.axis_index("c")
    s = lax.axis_index("s")
    wid = s * NC + c
    nj = NJ0 + jnp.where(wid < NXT, 1, 0)
    base = NJ0 * wid + jnp.minimum(wid, NXT)
    pltpu.sync_copy(edge_hbm.at[0, pl.ds(base, NJ0)],
                    sidx.at[pl.ds(0, NJ0)])
    pltpu.sync_copy(edge_hbm.at[1, pl.ds(base, NJ0)],
                    didx.at[pl.ds(0, NJ0)])

    @pl.when(wid < NXT)
    def _extra():
        pltpu.sync_copy(edge_hbm.at[0, pl.ds(base + NJ0, 1)],
                        sidx.at[pl.ds(NJ0, 1)])
        pltpu.sync_copy(edge_hbm.at[1, pl.ds(base + NJ0, 1)],
                        didx.at[pl.ds(NJ0, 1)])

    rows_slc = pl.ds(s * RPT, RPT)
    pltpu.sync_copy(rows_hbm.at[rows_slc], tbl.at[rows_slc])
    pltpu.sync_copy(z32_hbm.at[rows_slc], acc.at[rows_slc])
    plsc.subcore_barrier()
    nh = nj // 2
    for b in range(NSLOT):
        pltpu.async_copy(tbl.at[sidx.at[b]], rbs[b], gsems[b])

    def body(i, carry):
        j0 = NSLOT * i
        for b in range(NSLOT):
            pltpu.make_async_copy(tbl.at[sidx.at[j0 + b]], rbs[b],
                                  gsems[b]).wait()
            pltpu.async_copy(rbs[b], acc.at[didx.at[j0 + b]], ssems[b],
                             add=True)

        @pl.when(i + 1 < nh)
        def _refill():
            for b in range(NSLOT):
                pltpu.make_async_copy(rbs[b], acc.at[didx.at[j0 + b]],
                                      ssems[b]).wait()
                pltpu.async_copy(tbl.at[sidx.at[j0 + NSLOT + b]], rbs[b],
                                 gsems[b])

        return carry

    lax.fori_loop(0, nh, body, 0)
    for b in range(NSLOT):
        pltpu.make_async_copy(rbs[b], acc.at[didx.at[2 * nh - NSLOT + b]],
                              ssems[b]).wait()

    @pl.when(nj > 2 * nh)
    def _tail():
        pltpu.async_copy(tbl.at[sidx.at[2 * nh]], rbs[0], gsems[0]).wait()
        pltpu.sync_copy(rbs[0], acc.at[didx.at[2 * nh]], add=True)

    plsc.subcore_barrier()
    pltpu.sync_copy(acc.at[rows_slc], out_hbm.at[c, rows_slc])


@functools.cache
def _prop_call():
    mesh = plsc.VectorSubcoreMesh(core_axis_name="c", subcore_axis_name="s",
                                  num_cores=NC, num_subcores=NS)
    return pl.kernel(
        _prop_body,
        out_type=jax.ShapeDtypeStruct((NC, NP, 32), jnp.float32),
        mesh=mesh,
        compiler_params=pltpu.CompilerParams(use_tc_tiling_on_sc=False),
        scratch_types=[
            pltpu.VMEM((KT, CH), jnp.int32),
            pltpu.VMEM((KT, CH), jnp.int32),
            [pltpu.VMEM((CH, 32), jnp.float32) for _ in range(NSLOT)],
            pltpu.VMEM_SHARED((NP, 32), jnp.float32),
            pltpu.VMEM_SHARED((NP, 32), jnp.float32),
            [pltpu.SemaphoreType.DMA for _ in range(NSLOT)],
            [pltpu.SemaphoreType.DMA for _ in range(NSLOT)],
        ],
    )


def _mm1_body(x_ref, w_ref, o_ref):
    o_ref[0:N, :] = jnp.dot(x_ref[...], w_ref[...],
                            preferred_element_type=jnp.float32)
    o_ref[N:NP, :] = jnp.zeros((NP - N, 2 * OUT_C), jnp.float32)


def _scale_body(cnt_ref, h0_ref, hs_ref, dinv_ref):
    deg = cnt_ref[0, :, 0:1] + cnt_ref[1, :, 0:1] + 1.0
    dinv = lax.rsqrt(deg)
    dinv_ref[...] = dinv * jnp.ones((1, 2 * OUT_C), jnp.float32)
    hs_ref[...] = h0_ref[...] * dinv


def _layer1_body(acc_ref, dinv_ref, hs_ref, b1_ref, w23_ref, gs_ref):
    ph = dinv_ref[...] * (acc_ref[0] + acc_ref[1] + hs_ref[...]) \
        + b1_ref[...]
    h = jnp.maximum(ph, 0.0)
    g = jnp.dot(h, w23_ref[...], preferred_element_type=jnp.float32)
    gs_ref[...] = g * dinv_ref[...]


def _final_body(acc_ref, dinv_ref, gs_ref, b23_ref, eps_ref, z_ref):
    pg = dinv_ref[...] * (acc_ref[0] + acc_ref[1] + gs_ref[...]) \
        + b23_ref[...]
    mu = pg[:N, :OUT_C]
    ls = jnp.minimum(pg[:N, OUT_C:], 10.0)
    z_ref[...] = mu + eps_ref[...] * jnp.exp(ls)


def kernel(x, edge_index, W1, b1, W2, b2, W3, b3):
    f32 = jnp.float32
    er = edge_index.reshape(2, EC, CH)
    z8 = jnp.zeros((NP, DW), f32)
    ones8 = jnp.concatenate(
        [jnp.ones((CH, 1), f32), jnp.zeros((CH, DW - 1), f32)], axis=1)
    z32 = jnp.zeros((NP, 32), f32)
    w23 = jnp.concatenate([W2, W3], axis=1)
    b23 = jnp.concatenate([b2, b3]).reshape(1, 32)
    b1r = b1.reshape(1, 32)

    counts = _deg_call()(er, ones8, z8)

    h0 = pl.pallas_call(
        _mm1_body,
        out_shape=jax.ShapeDtypeStruct((NP, 32), f32),
    )(x, W1)

    hs, dinv32 = pl.pallas_call(
        _scale_body,
        out_shape=[
            jax.ShapeDtypeStruct((NP, 32), f32),
            jax.ShapeDtypeStruct((NP, 32), f32),
        ],
    )(counts, h0)

    acc1 = _prop_call()(hs, er, z32)

    gs = pl.pallas_call(
        _layer1_body,
        out_shape=jax.ShapeDtypeStruct((NP, 32), f32),
    )(acc1, dinv32, hs, b1r, w23)

    acc2 = _prop_call()(gs, er, z32)

    z = pl.pallas_call(
        _final_body,
        out_shape=jax.ShapeDtypeStruct((N, OUT_C), f32),
    )(acc2, dinv32, gs, b23, _EPS)
    return z


# speedup vs baseline: 1.2367x; 1.0467x over previous
import functools

import jax
import jax.numpy as jnp
from jax import lax
from jax.experimental import pallas as pl
from jax.experimental.pallas import tpu as pltpu
from jax.experimental.pallas import tpu_sc as plsc

N = 10000
E = 320000
IN_C = 128
OUT_C = 16

NC = 2
NS = 16
NW = NC * NS
CH = 128
EC = E // CH
NJ0 = EC // NW
NXT = EC - NW * NJ0
KT = NJ0 + 1
NP = 10112
RPT = NP // NS

_EPS = jax.random.normal(jax.random.key(42), (N, OUT_C), dtype=jnp.float32)


DW = 8


def _deg_body(edge_hbm, ones_hbm, z8_hbm, out_hbm, idx_v, ones_v, acc, sem):
    c = lax.axis_index("c")
    s = lax.axis_index("s")
    wid = s * NC + c
    nj = NJ0 + jnp.where(wid < NXT, 1, 0)
    base = NJ0 * wid + jnp.minimum(wid, NXT)
    pltpu.sync_copy(ones_hbm, ones_v)
    pltpu.sync_copy(z8_hbm.at[pl.ds(s * RPT, RPT)], acc.at[pl.ds(s * RPT, RPT)])
    pltpu.sync_copy(edge_hbm.at[1, pl.ds(base, NJ0)], idx_v.at[pl.ds(0, NJ0)])

    @pl.when(wid < NXT)
    def _extra():
        pltpu.sync_copy(edge_hbm.at[1, pl.ds(base + NJ0, 1)],
                        idx_v.at[pl.ds(NJ0, 1)])

    plsc.subcore_barrier()

    def body(j, carry):
        pltpu.async_copy(ones_v, acc.at[idx_v.at[j]], sem, add=True)
        return carry

    lax.fori_loop(0, nj, body, 0)

    def drain(j, carry):
        pltpu.make_async_copy(ones_v, acc.at[idx_v.at[j]], sem).wait()
        return carry

    lax.fori_loop(0, nj, drain, 0)
    plsc.subcore_barrier()
    pltpu.sync_copy(acc.at[pl.ds(s * RPT, RPT)],
                    out_hbm.at[c, pl.ds(s * RPT, RPT)])


@functools.cache
def _deg_call():
    mesh = plsc.VectorSubcoreMesh(core_axis_name="c", subcore_axis_name="s",
                                  num_cores=NC, num_subcores=NS)
    return pl.kernel(
        _deg_body,
        out_type=jax.ShapeDtypeStruct((NC, NP, DW), jnp.float32),
        mesh=mesh,
        compiler_params=pltpu.CompilerParams(use_tc_tiling_on_sc=False),
        scratch_types=[
            pltpu.VMEM((KT, CH), jnp.int32),
            pltpu.VMEM((CH, DW), jnp.float32),
            pltpu.VMEM_SHARED((NP, DW), jnp.float32),
            pltpu.SemaphoreType.DMA,
        ],
    )


NSLOT = 2


def _prop_body(rows_hbm, edge_hbm, z32_hbm, out_hbm,
               sidx, didx, rbs, acc, tbl, gsems, ssems):
    c = lax.axis_index("c")
    s = lax.axis_index("s")
    wid = s * NC + c
    nj = NJ0 + jnp.where(wid < NXT, 1, 0)
    base = NJ0 * wid + jnp.minimum(wid, NXT)
    pltpu.sync_copy(edge_hbm.at[0, pl.ds(base, NJ0)],
                    sidx.at[pl.ds(0, NJ0)])
    pltpu.sync_copy(edge_hbm.at[1, pl.ds(base, NJ0)],
                    didx.at[pl.ds(0, NJ0)])

    @pl.when(wid < NXT)
    def _extra():
        pltpu.sync_copy(edge_hbm.at[0, pl.ds(base + NJ0, 1)],
                        sidx.at[pl.ds(NJ0, 1)])
        pltpu.sync_copy(edge_hbm.at[1, pl.ds(base + NJ0, 1)],
                        didx.at[pl.ds(NJ0, 1)])

    rows_slc = pl.ds(s * RPT, RPT)
    pltpu.sync_copy(rows_hbm.at[rows_slc], tbl.at[rows_slc])
    pltpu.sync_copy(z32_hbm.at[rows_slc], acc.at[rows_slc])
    plsc.subcore_barrier()
    nh = nj // 2
    for b in range(NSLOT):
        pltpu.async_copy(tbl.at[sidx.at[b]], rbs[b], gsems[b])

    def body(i, carry):
        j0 = NSLOT * i
        for b in range(NSLOT):
            pltpu.make_async_copy(tbl.at[sidx.at[j0 + b]], rbs[b],
                                  gsems[b]).wait()
            pltpu.async_copy(rbs[b], acc.at[didx.at[j0 + b]], ssems[b],
                             add=True)

        @pl.when(i + 1 < nh)
        def _refill():
            for b in range(NSLOT):
                pltpu.make_async_copy(rbs[b], acc.at[didx.at[j0 + b]],
                                      ssems[b]).wait()
                pltpu.async_copy(tbl.at[sidx.at[j0 + NSLOT + b]], rbs[b],
                                 gsems[b])

        return carry

    lax.fori_loop(0, nh, body, 0)
    for b in range(NSLOT):
        pltpu.make_async_copy(rbs[b], acc.at[didx.at[2 * nh - NSLOT + b]],
                              ssems[b]).wait()

    @pl.when(nj > 2 * nh)
    def _tail():
        pltpu.async_copy(tbl.at[sidx.at[2 * nh]], rbs[0], gsems[0]).wait()
        pltpu.sync_copy(rbs[0], acc.at[didx.at[2 * nh]], add=True)

    plsc.subcore_barrier()
    pltpu.sync_copy(acc.at[rows_slc], out_hbm.at[c, rows_slc])


@functools.cache
def _prop_call():
    mesh = plsc.VectorSubcoreMesh(core_axis_name="c", subcore_axis_name="s",
                                  num_cores=NC, num_subcores=NS)
    return pl.kernel(
        _prop_body,
        out_type=jax.ShapeDtypeStruct((NC, NP, 32), jnp.float32),
        mesh=mesh,
        compiler_params=pltpu.CompilerParams(use_tc_tiling_on_sc=False),
        scratch_types=[
            pltpu.VMEM((KT, CH), jnp.int32),
            pltpu.VMEM((KT, CH), jnp.int32),
            [pltpu.VMEM((CH, 32), jnp.float32) for _ in range(NSLOT)],
            pltpu.VMEM_SHARED((NP, 32), jnp.float32),
            pltpu.VMEM_SHARED((NP, 32), jnp.float32),
            [pltpu.SemaphoreType.DMA for _ in range(NSLOT)],
            [pltpu.SemaphoreType.DMA for _ in range(NSLOT)],
        ],
    )


def _mm1_body(x_ref, w_ref, o_ref):
    o_ref[0:N, :] = jnp.dot(x_ref[...], w_ref[...],
                            preferred_element_type=jnp.float32)
    o_ref[N:NP, :] = jnp.zeros((NP - N, 2 * OUT_C), jnp.float32)


def _scale_body(cnt_ref, h0_ref, hs_ref, dinv_ref):
    deg = cnt_ref[0, :, 0:1] + cnt_ref[1, :, 0:1] + 1.0
    dinv = lax.rsqrt(deg)
    dinv_ref[...] = dinv * jnp.ones((1, 2 * OUT_C), jnp.float32)
    hs_ref[...] = h0_ref[...] * dinv


def _layer1_body(acc_ref, dinv_ref, hs_ref, b1_ref, w23_ref, gs_ref):
    ph = dinv_ref[...] * (acc_ref[0] + acc_ref[1] + hs_ref[...]) \
        + b1_ref[...]
    h = jnp.maximum(ph, 0.0)
    g = jnp.dot(h, w23_ref[...], preferred_element_type=jnp.float32)
    gs_ref[...] = g * dinv_ref[...]


def _final_body(acc_ref, dinv_ref, gs_ref, b23_ref, eps_ref, z_ref):
    pg = dinv_ref[...] * (acc_ref[0] + acc_ref[1] + gs_ref[...]) \
        + b23_ref[...]
    mu = pg[:N, :OUT_C]
    ls = jnp.minimum(pg[:N, OUT_C:], 10.0)
    z_ref[...] = mu + eps_ref[...] * jnp.exp(ls)


def kernel(x, edge_index, W1, b1, W2, b2, W3, b3):
    f32 = jnp.float32
    er = edge_index.reshape(2, EC, CH)
    z8 = jnp.zeros((NP, DW), f32)
    ones8 = jnp.concatenate(
        [jnp.ones((CH, 1), f32), jnp.zeros((CH, DW - 1), f32)], axis=1)
    z32 = jnp.zeros((NP, 32), f32)
    w23 = jnp.concatenate([W2, W3], axis=1)
    b23 = jnp.concatenate([b2, b3]).reshape(1, 32)
    b1r = b1.reshape(1, 32)

    counts = _deg_call()(er, ones8, z8)

    h0 = pl.pallas_call(
        _mm1_body,
        out_shape=jax.ShapeDtypeStruct((NP, 32), f32),
    )(x, W1)

    hs, dinv32 = pl.pallas_call(
        _scale_body,
        out_shape=[
            jax.ShapeDtypeStruct((NP, 32), f32),
            jax.ShapeDtypeStruct((NP, 32), f32),
        ],
    )(counts, h0)

    acc1 = _prop_call()(hs, er, z32)

    gs = pl.pallas_call(
        _layer1_body,
        out_shape=jax.ShapeDtypeStruct((NP, 32), f32),
    )(acc1, dinv32, hs, b1r, w23)

    acc2 = _prop_call()(gs, er, z32)

    z = pl.pallas_call(
        _final_body,
        out_shape=jax.ShapeDtypeStruct((N, OUT_C), f32),
    )(acc2, dinv32, gs, b23, _EPS)
    return z
